# baseline (speedup 1.0000x reference)
import functools

import jax
import jax.numpy as jnp
from jax import lax
from jax.experimental import pallas as pl
from jax.experimental.pallas import tpu as pltpu
from jax.experimental.pallas import tpu_sc as plsc

_B = 256
_N = 128
_H = 8
_V = 129
_ROWS = 32
_CHUNK = _ROWS * _N
_VECS = _CHUNK // 16
_NCHUNK = _N // _ROWS


@functools.cache
def _build_sc_kernel():
    info = plsc.get_sparse_core_info()
    nc, ns = info.num_cores, info.num_subcores
    nw = nc * ns
    bpw = _B // nw
    mesh = plsc.VectorSubcoreMesh(core_axis_name="c", subcore_axis_name="s")

    @functools.partial(
        pl.kernel,
        mesh=mesh,
        out_type=jax.ShapeDtypeStruct((_B, _H, _N, _N), jnp.float32),
        compiler_params=pltpu.CompilerParams(needs_layout_passes=False),
        scratch_types=[
            pltpu.VMEM((_V, _H), jnp.float32),
            pltpu.VMEM((2, _ROWS, _N), jnp.int32),
            pltpu.VMEM((2, _H, _ROWS, _N), jnp.float32),
            pltpu.SemaphoreType.DMA((2,)),
            pltpu.SemaphoreType.DMA((2,)),
        ],
    )
    def sc_kernel(dm_hbm, tab_hbm, out_hbm, tab_v, idx_v, out_v, in_sem, out_sem):
        wid = lax.axis_index("s") * nc + lax.axis_index("c")
        pltpu.sync_copy(tab_hbm, tab_v)
        steps = [(bi, ci) for bi in range(bpw) for ci in range(_NCHUNK)]
        nst = len(steps)
        hvecs = [jnp.full((16,), h, jnp.int32) for h in range(_H)]

        def in_copy(t, buf):
            bi, ci = steps[t]
            b = wid * bpw + bi
            return pltpu.async_copy(
                dm_hbm.at[b, pl.ds(ci * _ROWS, _ROWS), :],
                idx_v.at[buf],
                in_sem.at[buf],
            )

        def out_copies(t, buf):
            bi, ci = steps[t]
            b = wid * bpw + bi
            return [
                pltpu.async_copy(
                    out_v.at[buf, h],
                    out_hbm.at[b, h, pl.ds(ci * _ROWS, _ROWS), :],
                    out_sem.at[buf],
                )
                for h in range(_H)
            ]

        pending = {}
        ic = in_copy(0, 0)
        for t in range(nst):
            cur = t & 1
            nxt_ic = in_copy(t + 1, 1 - cur) if t + 1 < nst else None
            ic.wait()
            if t >= 2:
                for c in pending.pop(t - 2):
                    c.wait()

            @plsc.parallel_loop(0, _VECS, unroll=2)
            def body(v):
                r = v // (_N // 16)
                c = (v % (_N // 16)) * 16
                idx = idx_v[cur, r, pl.ds(c, 16)]
                for h in range(_H):
                    val = plsc.load_gather(tab_v, [idx, hvecs[h]])
                    out_v[cur, h, r, pl.ds(c, 16)] = val

            pending[t] = out_copies(t, cur)
            ic = nxt_ic
        for t in (nst - 2, nst - 1):
            for c in pending.pop(t):
                c.wait()

    return sc_kernel


def kernel(distance_matrix, distance_embedding):
    dm = distance_matrix
    if dm.dtype != jnp.int32:
        dm = dm.astype(jnp.int32)
    return _build_sc_kernel()(dm, distance_embedding)
